# Initial kernel scaffold; baseline (speedup 1.0000x reference)
import jax
import jax.numpy as jnp
from jax.experimental import pallas as pl


def kernel(x, weights, filters):
    raise NotImplementedError("write your pallas kernel here")


# speedup vs baseline: 4.1764x; 4.1764x over previous
import jax
import jax.numpy as jnp
from jax.experimental import pallas as pl
from jax.experimental.pallas import tpu as pltpu


def _fcl_body(x_ref, f_ref, o_ref):
    _, cin, hw = x_ref.shape
    kk = f_ref.shape[0]
    k = int(round(kk ** 0.5))
    w = int(round(hw ** 0.5))
    h = hw // w
    xb = x_ref[0].astype(jnp.bfloat16)

    lane = jax.lax.broadcasted_iota(jnp.int32, (1, hw), 1)
    oh = lane // w
    ow = lane - oh * w

    acc = None
    for kh in range(k):
        for kw in range(k):
            t = kh * k + kw
            dh, dw = kh - 1, kw - 1
            off = dh * w + dw
            if off == 0:
                xs = xb
            else:
                xs = jnp.concatenate([xb[:, off:], xb[:, :off]], axis=1)
            conds = []
            if dh == -1:
                conds.append(oh >= 1)
            elif dh == 1:
                conds.append(oh <= h - 2)
            if dw == -1:
                conds.append(ow >= 1)
            elif dw == 1:
                conds.append(ow <= w - 2)
            if conds:
                valid = conds[0]
                for c in conds[1:]:
                    valid = jnp.logical_and(valid, c)
                xs = jnp.where(valid, xs, jnp.zeros_like(xs))
            d = jnp.dot(f_ref[t], xs, preferred_element_type=jnp.float32)
            acc = d if acc is None else acc + d
    o_ref[0] = acc


def kernel(x, weights, filters):
    n, cin, h, w = x.shape
    cout, _, f = weights.shape
    k = filters.shape[-1]
    kk = k * k
    hw = h * w

    fmat = jnp.einsum("oif,fp->poi",
                      weights.astype(jnp.float32),
                      filters.reshape(f, kk).astype(jnp.float32))
    fmat = fmat.astype(jnp.bfloat16)

    xf = x.reshape(n, cin, hw)

    cost = pl.CostEstimate(
        flops=2 * n * hw * cin * kk * cout,
        transcendentals=0,
        bytes_accessed=xf.size * 4 + fmat.size * 2 + n * cout * hw * 4,
    )

    out = pl.pallas_call(
        _fcl_body,
        out_shape=jax.ShapeDtypeStruct((n, cout, hw), jnp.float32),
        grid=(n,),
        in_specs=[
            pl.BlockSpec((1, cin, hw), lambda g: (g, 0, 0)),
            pl.BlockSpec((kk, cout, cin), lambda g: (0, 0, 0)),
        ],
        out_specs=pl.BlockSpec((1, cout, hw), lambda g: (g, 0, 0)),
        compiler_params=pltpu.CompilerParams(
            dimension_semantics=("parallel",),
        ),
        cost_estimate=cost,
    )(xf, fmat)

    return out.reshape(n, cout, h, w)


# speedup vs baseline: 5.0599x; 1.2115x over previous
import jax
import jax.numpy as jnp
from jax.experimental import pallas as pl
from jax.experimental.pallas import tpu as pltpu


def _fcl_body(x_ref, f_ref, o_ref):
    b, cin, hw = x_ref.shape
    kk = f_ref.shape[0]
    k = int(round(kk ** 0.5))
    w = int(round(hw ** 0.5))
    h = hw // w

    lane = jax.lax.broadcasted_iota(jnp.int32, (1, hw), 1)
    oh = lane // w
    ow = lane - oh * w
    masks = {}
    for kh in range(k):
        for kw in range(k):
            dh, dw = kh - 1, kw - 1
            conds = []
            if dh == -1:
                conds.append(oh >= 1)
            elif dh == 1:
                conds.append(oh <= h - 2)
            if dw == -1:
                conds.append(ow >= 1)
            elif dw == 1:
                conds.append(ow <= w - 2)
            if conds:
                valid = conds[0]
                for c in conds[1:]:
                    valid = jnp.logical_and(valid, c)
                masks[(kh, kw)] = valid

    for bi in range(b):
        xb = x_ref[bi].astype(jnp.bfloat16)
        acc = None
        for kh in range(k):
            for kw in range(k):
                t = kh * k + kw
                off = (kh - 1) * w + (kw - 1)
                if off == 0:
                    xs = xb
                else:
                    xs = jnp.concatenate([xb[:, off:], xb[:, :off]], axis=1)
                if (kh, kw) in masks:
                    xs = jnp.where(masks[(kh, kw)], xs, jnp.zeros_like(xs))
                d = jnp.dot(f_ref[t], xs,
                            preferred_element_type=jnp.float32)
                acc = d if acc is None else acc + d
        o_ref[bi] = acc


def kernel(x, weights, filters):
    n, cin, h, w = x.shape
    cout, _, f = weights.shape
    k = filters.shape[-1]
    kk = k * k
    hw = h * w

    fmat = jnp.einsum("oif,fp->poi",
                      weights.astype(jnp.float32),
                      filters.reshape(f, kk).astype(jnp.float32))
    fmat = fmat.astype(jnp.bfloat16)

    xf = x.reshape(n, cin, hw)

    cost = pl.CostEstimate(
        flops=2 * n * hw * cin * kk * cout,
        transcendentals=0,
        bytes_accessed=xf.size * 4 + fmat.size * 2 + n * cout * hw * 4,
    )

    blk = 8 if n % 8 == 0 else 1
    out = pl.pallas_call(
        _fcl_body,
        out_shape=jax.ShapeDtypeStruct((n, cout, hw), jnp.float32),
        grid=(n // blk,),
        in_specs=[
            pl.BlockSpec((blk, cin, hw), lambda g: (g, 0, 0)),
            pl.BlockSpec((kk, cout, cin), lambda g: (0, 0, 0)),
        ],
        out_specs=pl.BlockSpec((blk, cout, hw), lambda g: (g, 0, 0)),
        compiler_params=pltpu.CompilerParams(
            dimension_semantics=("parallel",),
        ),
        cost_estimate=cost,
    )(xf, fmat)

    return out.reshape(n, cout, h, w)
